# Initial kernel scaffold; baseline (speedup 1.0000x reference)
import jax
import jax.numpy as jnp
from jax.experimental import pallas as pl


def kernel(x, edge_index, W_in, b_in, ln_g, ln_b, W_neigh, W_self, b_conv, W_out, b_out):
    raise NotImplementedError("write your pallas kernel here")


# speedup vs baseline: 4.5515x; 4.5515x over previous
import functools

import jax
import jax.numpy as jnp
from jax import lax
from jax.experimental import pallas as pl
from jax.experimental.pallas import tpu as pltpu
from jax.experimental.pallas import tpu_sc as plsc

N = 10000
E = 320000
D = 128
DOUT = 64
NC = 2
NS = 16
NW = NC * NS
CHUNK = 128
NCHUNK = -(-E // (NW * CHUNK))
EPAD = NW * NCHUNK * CHUNK
NPAD = 10112
RPT = NPAD // NS
BR = 2000

@functools.cache
def _mesh():
    return plsc.VectorSubcoreMesh(core_axis_name="c", subcore_axis_name="s",
                                  num_cores=NC, num_subcores=NS)


def _edge_body(src_hbm, dst_hbm, z_hbm, zeros_hbm, out_hbm,
               src_v, dst_v, rows_v, acc, sem):
    c = lax.axis_index("c")
    s = lax.axis_index("s")
    wid = s * NC + c
    pltpu.sync_copy(zeros_hbm.at[pl.ds(s * RPT, RPT)],
                    acc.at[pl.ds(s * RPT, RPT)])
    pltpu.sync_copy(src_hbm.at[wid], src_v)
    pltpu.sync_copy(dst_hbm.at[wid], dst_v)
    plsc.subcore_barrier()

    def step(j, carry):
        pltpu.async_copy(z_hbm.at[src_v.at[j]], rows_v, sem).wait()
        pltpu.sync_copy(rows_v, acc.at[dst_v.at[j]], add=True)
        return carry

    lax.fori_loop(0, NCHUNK, step, 0)
    plsc.subcore_barrier()
    pltpu.sync_copy(acc.at[pl.ds(s * RPT, RPT)],
                    out_hbm.at[c, pl.ds(s * RPT, RPT)])


@functools.cache
def _edge_kernel():
    return pl.kernel(
        _edge_body,
        out_type=jax.ShapeDtypeStruct((NC, NPAD, D), jnp.float32),
        mesh=_mesh(),
        scratch_types=[
            pltpu.VMEM((NCHUNK, CHUNK), jnp.int32),
            pltpu.VMEM((NCHUNK, CHUNK), jnp.int32),
            pltpu.VMEM((CHUNK, D), jnp.float32),
            pltpu.VMEM_SHARED((NPAD, D), jnp.float32),
            pltpu.SemaphoreType.DMA,
        ],
    )


def _deg_body(dst_hbm, ones_hbm, zeros_hbm, out_hbm, dst_v, ones_v, acc):
    c = lax.axis_index("c")
    s = lax.axis_index("s")
    wid = s * NC + c
    pltpu.sync_copy(zeros_hbm.at[pl.ds(s * RPT, RPT)],
                    acc.at[pl.ds(s * RPT, RPT)])
    pltpu.sync_copy(ones_hbm, ones_v)
    pltpu.sync_copy(dst_hbm.at[wid], dst_v)
    plsc.subcore_barrier()

    def step(j, carry):
        pltpu.sync_copy(ones_v, acc.at[dst_v.at[j]], add=True)
        return carry

    lax.fori_loop(0, NCHUNK, step, 0)
    plsc.subcore_barrier()
    pltpu.sync_copy(acc.at[pl.ds(s * RPT, RPT)],
                    out_hbm.at[c, pl.ds(s * RPT, RPT)])


@functools.cache
def _deg_kernel():
    return pl.kernel(
        _deg_body,
        out_type=jax.ShapeDtypeStruct((NC, NPAD, D), jnp.float32),
        mesh=_mesh(),
        scratch_types=[
            pltpu.VMEM((NCHUNK, CHUNK), jnp.int32),
            pltpu.VMEM((CHUNK, D), jnp.float32),
            pltpu.VMEM_SHARED((NPAD, D), jnp.float32),
        ],
    )


def _ln(h, g, b):
    mu = jnp.mean(h, axis=-1, keepdims=True)
    var = jnp.mean((h - mu) ** 2, axis=-1, keepdims=True)
    return (h - mu) * lax.rsqrt(var + 1e-5) * g + b


def _proj_body(x_ref, w_ref, b_ref, g_ref, bb_ref, h_ref, z_ref):
    h = jnp.dot(x_ref[...], w_ref[...],
                preferred_element_type=jnp.float32) + b_ref[...]
    h_ref[...] = h
    z_ref[...] = _ln(h, g_ref[...], bb_ref[...])


_proj_ln = pl.pallas_call(
    _proj_body,
    grid=(N // BR,),
    in_specs=[
        pl.BlockSpec((BR, D), lambda i: (i, 0)),
        pl.BlockSpec((D, D), lambda i: (0, 0)),
        pl.BlockSpec((1, D), lambda i: (0, 0)),
        pl.BlockSpec((1, D), lambda i: (0, 0)),
        pl.BlockSpec((1, D), lambda i: (0, 0)),
    ],
    out_specs=[pl.BlockSpec((BR, D), lambda i: (i, 0)),
               pl.BlockSpec((BR, D), lambda i: (i, 0))],
    out_shape=[jax.ShapeDtypeStruct((N, D), jnp.float32),
               jax.ShapeDtypeStruct((N, D), jnp.float32)],
)


def _sage_common(h_ref, z_ref, p_ref, dg_ref, wn_ref, ws_ref, bc_ref):
    deg = jnp.maximum(dg_ref[0, :, 0:1] + dg_ref[1, :, 0:1], 1.0)
    agg = (p_ref[0] + p_ref[1]) / deg
    conv = (jnp.dot(agg, wn_ref[...], preferred_element_type=jnp.float32)
            + jnp.dot(z_ref[...], ws_ref[...], preferred_element_type=jnp.float32)
            + bc_ref[...])
    return jnp.maximum(h_ref[...] + conv, 0.0)


def _mid_body(h_ref, z_ref, p_ref, dg_ref, wn_ref, ws_ref, bc_ref,
              g_ref, bb_ref, ho_ref, zo_ref):
    hn = _sage_common(h_ref, z_ref, p_ref, dg_ref, wn_ref, ws_ref, bc_ref)
    ho_ref[...] = hn
    zo_ref[...] = _ln(hn, g_ref[...], bb_ref[...])


_SAGE_SPECS = [
    pl.BlockSpec((BR, D), lambda i: (i, 0)),
    pl.BlockSpec((BR, D), lambda i: (i, 0)),
    pl.BlockSpec((NC, BR, D), lambda i: (0, i, 0)),
    pl.BlockSpec((NC, BR, D), lambda i: (0, i, 0)),
    pl.BlockSpec((D, D), lambda i: (0, 0)),
    pl.BlockSpec((D, D), lambda i: (0, 0)),
    pl.BlockSpec((1, D), lambda i: (0, 0)),
]

_mid_layer = pl.pallas_call(
    _mid_body,
    grid=(N // BR,),
    in_specs=_SAGE_SPECS + [
        pl.BlockSpec((1, D), lambda i: (0, 0)),
        pl.BlockSpec((1, D), lambda i: (0, 0)),
    ],
    out_specs=[pl.BlockSpec((BR, D), lambda i: (i, 0)),
               pl.BlockSpec((BR, D), lambda i: (i, 0))],
    out_shape=[jax.ShapeDtypeStruct((N, D), jnp.float32),
               jax.ShapeDtypeStruct((N, D), jnp.float32)],
)


def _last_body(h_ref, z_ref, p_ref, dg_ref, wn_ref, ws_ref, bc_ref,
               wo_ref, bo_ref, o_ref):
    hn = _sage_common(h_ref, z_ref, p_ref, dg_ref, wn_ref, ws_ref, bc_ref)
    o_ref[...] = jnp.dot(hn, wo_ref[...],
                         preferred_element_type=jnp.float32) + bo_ref[...]


_last_layer = pl.pallas_call(
    _last_body,
    grid=(N // BR,),
    in_specs=_SAGE_SPECS + [
        pl.BlockSpec((D, DOUT), lambda i: (0, 0)),
        pl.BlockSpec((1, DOUT), lambda i: (0, 0)),
    ],
    out_specs=pl.BlockSpec((BR, DOUT), lambda i: (i, 0)),
    out_shape=jax.ShapeDtypeStruct((N, DOUT), jnp.float32),
)


def kernel(x, edge_index, W_in, b_in, ln_g, ln_b, W_neigh, W_self, b_conv,
           W_out, b_out):
    i32 = jnp.int32
    src = edge_index[0].astype(i32)
    dst = edge_index[1].astype(i32)
    src_p = jnp.concatenate([src, jnp.zeros((EPAD - E,), i32)])
    dst_p = jnp.concatenate([dst, jnp.full((EPAD - E,), N, i32)])
    src_p = src_p.reshape(NW, NCHUNK, CHUNK)
    dst_p = dst_p.reshape(NW, NCHUNK, CHUNK)

    zeros_d = jnp.zeros((NPAD, D), jnp.float32)
    ones_d = jnp.ones((CHUNK, D), jnp.float32)

    degp = _deg_kernel()(dst_p, ones_d, zeros_d)
    h, z = _proj_ln(x, W_in, b_in.reshape(1, D),
                    ln_g[0].reshape(1, D), ln_b[0].reshape(1, D))

    out = None
    for i in range(W_self.shape[0]):
        parts = _edge_kernel()(src_p, dst_p, z, zeros_d)
        if i + 1 < W_self.shape[0]:
            h, z = _mid_layer(h, z, parts, degp, W_neigh[i], W_self[i],
                              b_conv[i].reshape(1, D),
                              ln_g[i + 1].reshape(1, D),
                              ln_b[i + 1].reshape(1, D))
        else:
            out = _last_layer(h, z, parts, degp, W_neigh[i], W_self[i],
                              b_conv[i].reshape(1, D), W_out,
                              b_out.reshape(1, DOUT))
    return out
